# baseline (speedup 1.0000x reference)
import functools

import jax
import jax.numpy as jnp
from jax import lax
from jax.experimental import pallas as pl
from jax.experimental.pallas import tpu as pltpu
from jax.experimental.pallas import tpu_sc as plsc

_NC = 2
_NS = 16
_NW = _NC * _NS
_CHUNK = 128


_SEG = 25000


def _sc_linearize(user_t, item_t, D, N):
    n_seg = N // _SEG
    rows_per_w = D // _NW if D >= _NW else 1
    assert D % _NW == 0 or _NW % D == 0
    mesh = plsc.VectorSubcoreMesh(core_axis_name="c", subcore_axis_name="s")

    @functools.partial(
        pl.kernel,
        mesh=mesh,
        out_type=[
            jax.ShapeDtypeStruct((D * N,), jnp.float32),
            jax.ShapeDtypeStruct((D * N,), jnp.float32),
        ],
        scratch_types=[
            pltpu.SemaphoreType.DMA,
        ],
    )
    def k(utab_hbm, itab_hbm, uout_hbm, iout_hbm, sem):
        wid = lax.axis_index("s") * _NC + lax.axis_index("c")
        d0 = wid * rows_per_w
        for tab_hbm, out_hbm in ((utab_hbm, uout_hbm), (itab_hbm, iout_hbm)):
            for dd in range(rows_per_w):
                d = d0 + dd
                for s in range(n_seg):
                    c0 = s * _SEG
                    pltpu.make_async_copy(
                        tab_hbm.at[d, pl.ds(c0, _SEG)],
                        out_hbm.at[pl.ds(d * N + c0, _SEG)], sem).start()
            pltpu.make_async_copy(
                out_hbm.at[pl.ds(0, rows_per_w * N)],
                out_hbm.at[pl.ds(d0 * N, rows_per_w * N)], sem).wait()

    return k(user_t, item_t)


def _sc_gather(user_idx2d, item_idx2d, user_flat, item_flat, B, D, N):
    b_per_w = B // _NW
    n_chunks = b_per_w // _CHUNK
    n_req = D * n_chunks
    mesh = plsc.VectorSubcoreMesh(core_axis_name="c", subcore_axis_name="s")

    @functools.partial(
        pl.kernel,
        mesh=mesh,
        out_type=[
            jax.ShapeDtypeStruct((D, B), jnp.float32),
            jax.ShapeDtypeStruct((D, B), jnp.float32),
        ],
        scratch_types=[
            pltpu.VMEM((n_chunks, _CHUNK), jnp.int32),
            pltpu.VMEM((n_chunks, _CHUNK), jnp.int32),
            pltpu.VMEM((n_req, _CHUNK), jnp.int32),
            pltpu.VMEM((D, b_per_w), jnp.float32),
            pltpu.SemaphoreType.DMA,
        ],
    )
    def k(uidx_hbm, iidx_hbm, utab_hbm, itab_hbm, uout_hbm, iout_hbm,
          uidx_v, iidx_v, widx_v, buf_v, sem):
        wid = lax.axis_index("s") * _NC + lax.axis_index("c")
        base = wid * b_per_w
        crow = wid * n_chunks
        pltpu.sync_copy(uidx_hbm.at[pl.ds(crow, n_chunks)], uidx_v)
        pltpu.sync_copy(iidx_hbm.at[pl.ds(crow, n_chunks)], iidx_v)

        for tab_hbm, idx_v, out_hbm in (
            (utab_hbm, uidx_v, uout_hbm),
            (itab_hbm, iidx_v, iout_hbm),
        ):
            def fire(r, carry):
                d = r // n_chunks
                c = r - d * n_chunks
                doff = d * N
                for j in range(_CHUNK // 16):
                    sl = pl.ds(j * 16, 16)
                    widx_v[r, sl] = idx_v[c, sl] + doff
                pltpu.make_async_copy(
                    tab_hbm.at[widx_v.at[r]],
                    buf_v.at[d, pl.ds(c * _CHUNK, _CHUNK)], sem).start()
                return carry

            lax.fori_loop(0, n_req, fire, 0)
            pltpu.make_async_copy(
                out_hbm.at[:, pl.ds(0, b_per_w)], buf_v, sem).wait()
            pltpu.sync_copy(buf_v, out_hbm.at[:, pl.ds(base, b_per_w)])

    return k(user_idx2d, item_idx2d, user_flat, item_flat)


def _mlp_body(u_ref, i_ref, w1_ref, b1_ref, w2_ref, b2_ref,
              pw1_ref, pb1_ref, pw2_ref, pb2_ref, o_ref):
    xt = jnp.concatenate([u_ref[...], i_ref[...]], axis=0)
    ht = jnp.dot(w1_ref[...], xt, preferred_element_type=jnp.float32)
    ht = jnp.maximum(ht + b1_ref[...], 0.0)
    logits = jnp.sum(ht * w2_ref[...], axis=0, keepdims=True) + b2_ref[0, 0]
    a = jax.nn.sigmoid(logits)
    xw = xt * a
    pt = jnp.dot(pw1_ref[...], xw, preferred_element_type=jnp.float32)
    pt = jnp.maximum(pt + pb1_ref[...], 0.0)
    o_ref[...] = jnp.sum(pt * pw2_ref[...], axis=0) + pb2_ref[0, 0]


def _tc_mlp(ut, it, att_w1t, att_b1, att_w2, att_b2,
            pred_w1t, pred_b1, pred_w2, pred_b2):
    D, B = ut.shape
    BLK = 2048
    full = lambda s: pl.BlockSpec(s, lambda i: (0,) * len(s))
    return pl.pallas_call(
        _mlp_body,
        grid=(B // BLK,),
        in_specs=[
            pl.BlockSpec((D, BLK), lambda i: (0, i)),
            pl.BlockSpec((D, BLK), lambda i: (0, i)),
            full(att_w1t.shape),
            full(att_b1.shape),
            full(att_w2.shape),
            full(att_b2.shape),
            full(pred_w1t.shape),
            full(pred_b1.shape),
            full(pred_w2.shape),
            full(pred_b2.shape),
        ],
        out_specs=pl.BlockSpec((BLK,), lambda i: (i,)),
        out_shape=jax.ShapeDtypeStruct((B,), jnp.float32),
    )(ut, it, att_w1t, att_b1, att_w2, att_b2,
      pred_w1t, pred_b1, pred_w2, pred_b2)


def kernel(user_indices, item_indices, user_table, item_table,
           att_w1, att_b1, att_w2, att_b2,
           pred_w1, pred_b1, pred_w2, pred_b2):
    B = user_indices.shape[0]
    N, D = user_table.shape
    uidx = user_indices.astype(jnp.int32).reshape(B // _CHUNK, _CHUNK)
    iidx = item_indices.astype(jnp.int32).reshape(B // _CHUNK, _CHUNK)
    ut, it = _sc_gather(
        uidx, iidx,
        user_table.T.reshape(D * N),
        item_table.T.reshape(D * N),
        B, D, N)
    return _tc_mlp(
        ut, it,
        att_w1.T, att_b1.reshape(-1, 1),
        att_w2.reshape(-1, 1), att_b2.reshape(1, 1),
        pred_w1.T, pred_b1.reshape(-1, 1),
        pred_w2.reshape(-1, 1), pred_b2.reshape(1, 1),
    )


# speedup vs baseline: 10.4943x; 10.4943x over previous
import functools

import jax
import jax.numpy as jnp
from jax import lax
from jax.experimental import pallas as pl
from jax.experimental.pallas import tpu as pltpu
from jax.experimental.pallas import tpu_sc as plsc

_NC = 2
_NS = 16
_NW = _NC * _NS


def _sc_gather(user_idx, item_idx, user_table, item_table, B, D):
    b_per_w = B // _NW
    half = b_per_w // 2
    mesh = plsc.VectorSubcoreMesh(core_axis_name="c", subcore_axis_name="s")

    @functools.partial(
        pl.kernel,
        mesh=mesh,
        out_type=[
            jax.ShapeDtypeStruct((B, D), jnp.float32),
            jax.ShapeDtypeStruct((B, D), jnp.float32),
        ],
        scratch_types=[
            pltpu.SMEM((b_per_w,), jnp.int32),
            pltpu.SMEM((b_per_w,), jnp.int32),
            pltpu.VMEM_SHARED((_NS, b_per_w), jnp.int32),
            pltpu.VMEM_SHARED((_NS, b_per_w), jnp.int32),
            pltpu.SemaphoreType.DMA,
            pltpu.SemaphoreType.DMA,
            pltpu.SemaphoreType.DMA,
            pltpu.SemaphoreType.DMA,
        ],
    )
    def k(uidx_hbm, iidx_hbm, utab_hbm, itab_hbm, uout_hbm, iout_hbm,
          usmem, ismem, uidx_sp, iidx_sp, sem0, sem1, sem2, sem3):
        sid = lax.axis_index("s")
        wid = sid * _NC + lax.axis_index("c")
        base = wid * b_per_w
        pltpu.sync_copy(uidx_hbm.at[pl.ds(base, b_per_w)], uidx_sp.at[sid])
        pltpu.sync_copy(iidx_hbm.at[pl.ds(base, b_per_w)], iidx_sp.at[sid])
        pltpu.sync_copy(uidx_sp.at[sid], usmem)
        pltpu.sync_copy(iidx_sp.at[sid], ismem)

        def fire(kk, carry):
            k0 = kk * 2
            k1 = k0 + 1
            for k_, us, is_ in ((k0, sem0, sem1), (k1, sem2, sem3)):
                ur = usmem[k_]
                ir = ismem[k_]
                row = base + k_
                pltpu.make_async_copy(
                    utab_hbm.at[ur >> 3, pl.ds(ur & 7, 1), :],
                    uout_hbm.at[pl.ds(row, 1), :], us).start()
                pltpu.make_async_copy(
                    itab_hbm.at[ir >> 3, pl.ds(ir & 7, 1), :],
                    iout_hbm.at[pl.ds(row, 1), :], is_).start()
            return carry

        lax.fori_loop(0, half, fire, 0)
        pltpu.make_async_copy(
            uout_hbm.at[pl.ds(0, half)],
            uout_hbm.at[pl.ds(base, half)], sem0).wait()
        pltpu.make_async_copy(
            iout_hbm.at[pl.ds(0, half)],
            iout_hbm.at[pl.ds(base, half)], sem1).wait()
        pltpu.make_async_copy(
            uout_hbm.at[pl.ds(0, half)],
            uout_hbm.at[pl.ds(base, half)], sem2).wait()
        pltpu.make_async_copy(
            iout_hbm.at[pl.ds(0, half)],
            iout_hbm.at[pl.ds(base, half)], sem3).wait()

    return k(user_idx, item_idx, user_table, item_table)


def _mlp_body(u_ref, i_ref, w1_ref, b1_ref, w2_ref, b2_ref,
              pw1_ref, pb1_ref, pw2_ref, pb2_ref, o_ref):
    x = jnp.concatenate([u_ref[...], i_ref[...]], axis=1)
    h = jnp.dot(x, w1_ref[...], preferred_element_type=jnp.float32)
    h = jnp.maximum(h + b1_ref[...], 0.0)
    logits = jnp.sum(h * w2_ref[...], axis=1, keepdims=True) + b2_ref[0, 0]
    a = jax.nn.sigmoid(logits)
    xw = x * a
    p = jnp.dot(xw, pw1_ref[...], preferred_element_type=jnp.float32)
    p = jnp.maximum(p + pb1_ref[...], 0.0)
    o_ref[...] = jnp.sum(p * pw2_ref[...], axis=1) + pb2_ref[0, 0]


def _tc_mlp(u, it, att_w1, att_b1, att_w2, att_b2,
            pred_w1, pred_b1, pred_w2, pred_b2):
    B, D = u.shape
    BLK = 2048
    full = lambda s: pl.BlockSpec(s, lambda i: (0,) * len(s))
    return pl.pallas_call(
        _mlp_body,
        grid=(B // BLK,),
        in_specs=[
            pl.BlockSpec((BLK, D), lambda i: (i, 0)),
            pl.BlockSpec((BLK, D), lambda i: (i, 0)),
            full(att_w1.shape),
            full(att_b1.shape),
            full(att_w2.shape),
            full(att_b2.shape),
            full(pred_w1.shape),
            full(pred_b1.shape),
            full(pred_w2.shape),
            full(pred_b2.shape),
        ],
        out_specs=pl.BlockSpec((BLK,), lambda i: (i,)),
        out_shape=jax.ShapeDtypeStruct((B,), jnp.float32),
    )(u, it, att_w1, att_b1, att_w2, att_b2,
      pred_w1, pred_b1, pred_w2, pred_b2)


def kernel(user_indices, item_indices, user_table, item_table,
           att_w1, att_b1, att_w2, att_b2,
           pred_w1, pred_b1, pred_w2, pred_b2):
    B = user_indices.shape[0]
    N, D = user_table.shape
    uidx = user_indices.astype(jnp.int32)
    iidx = item_indices.astype(jnp.int32)
    u, it = _sc_gather(
        uidx, iidx,
        user_table.reshape(N // 8, 8, D),
        item_table.reshape(N // 8, 8, D),
        B, D)
    return _tc_mlp(
        u, it,
        att_w1, att_b1.reshape(1, -1),
        att_w2.reshape(1, -1), att_b2.reshape(1, 1),
        pred_w1, pred_b1.reshape(1, -1),
        pred_w2.reshape(1, -1), pred_b2.reshape(1, 1),
    )
